# baseline (speedup 1.0000x reference)
import functools

import jax
import jax.numpy as jnp
from jax import lax
from jax.experimental import pallas as pl
from jax.experimental.pallas import tpu as pltpu
from jax.experimental.pallas import tpu_sc as plsc

_LANES = 16
_NC = 2
_NS = 16
_NW = _NC * _NS
_BATCH = 16384
_D = 128
_ROWS_PER_W = _BATCH // _NW
_CHUNKS = (32, 96, 128, 128, 96, 32)
_MAXCHUNK = 128
_NBUF = 3
_VECS = _D // _LANES
_SCALE = 0.5 / (2.0 * _BATCH)


def _sc_center_loss_partials(features, labels, centers):
    mesh = plsc.VectorSubcoreMesh(core_axis_name="c", subcore_axis_name="s")

    @functools.partial(
        pl.kernel,
        out_type=jax.ShapeDtypeStruct((_NW, _LANES), jnp.float32),
        mesh=mesh,
        scratch_types=[
            pltpu.VMEM((_ROWS_PER_W,), jnp.int32),
            pltpu.VMEM((_NBUF, _MAXCHUNK, _D), jnp.float32),
            pltpu.VMEM((_NBUF, _MAXCHUNK, _D), jnp.float32),
            pltpu.VMEM((_LANES,), jnp.float32),
        ] + [pltpu.SemaphoreType.DMA] * (2 * _NBUF),
    )
    def run(feat_hbm, idx_hbm, cent_hbm, out_hbm,
            idx_v, cent_v, feat_v, acc_v, *sems):
        wid = lax.axis_index("s") * _NC + lax.axis_index("c")
        base = wid * _ROWS_PER_W
        sg = sems[:_NBUF]
        sf = sems[_NBUF:]
        offs = []
        o = 0
        for n in _CHUNKS:
            offs.append(o)
            o += n

        def start(c):
            b = c % _NBUF
            n = _CHUNKS[c]
            o = offs[c]
            g = pltpu.async_copy(
                cent_hbm.at[idx_v.at[pl.ds(o, n)]],
                cent_v.at[b, pl.ds(0, n)], sg[b])
            f = pltpu.async_copy(
                feat_hbm.at[pl.ds(base + o, n)],
                feat_v.at[b, pl.ds(0, n)], sf[b])
            return g, f

        n0 = _CHUNKS[0]
        pltpu.sync_copy(idx_hbm.at[pl.ds(base, n0)], idx_v.at[pl.ds(0, n0)])
        pending = [start(0)]
        pltpu.sync_copy(idx_hbm.at[pl.ds(base + n0, _ROWS_PER_W - n0)],
                        idx_v.at[pl.ds(n0, _ROWS_PER_W - n0)])
        pending += [start(c) for c in range(1, _NBUF)]
        acc = tuple(jnp.zeros((_LANES,), jnp.float32) for _ in range(_VECS))
        for c in range(len(_CHUNKS)):
            b = c % _NBUF
            g, f = pending[b]
            g.wait()
            f.wait()

            def body(r, a, b=b):
                out = []
                for v in range(_VECS):
                    fv = feat_v[b, r, pl.ds(v * _LANES, _LANES)]
                    cv = cent_v[b, r, pl.ds(v * _LANES, _LANES)]
                    d = fv - cv
                    out.append(a[v] + d * d)
                return tuple(out)

            acc = plsc.parallel_loop(0, _CHUNKS[c], unroll=4, carry=acc)(body)
            if c + _NBUF < len(_CHUNKS):
                pending[b] = start(c + _NBUF)

        tot = acc[0]
        for v in range(1, _VECS):
            tot = tot + acc[v]
        acc_v[...] = tot * jnp.float32(_SCALE)
        pltpu.sync_copy(acc_v, out_hbm.at[wid])

    return run(features, labels, centers)


def kernel(features, labels, centers):
    labels_i32 = labels.astype(jnp.int32)
    partials = _sc_center_loss_partials(features, labels_i32, centers)
    return jnp.sum(partials)


# speedup vs baseline: 1.0235x; 1.0235x over previous
import functools

import jax
import jax.numpy as jnp
from jax import lax
from jax.experimental import pallas as pl
from jax.experimental.pallas import tpu as pltpu
from jax.experimental.pallas import tpu_sc as plsc

_LANES = 16
_NC = 2
_NS = 16
_NW = _NC * _NS
_BATCH = 16384
_D = 128
_ROWS_PER_W = _BATCH // _NW
_CHUNKS = (32, 96, 128, 128, 96, 32)
_MAXCHUNK = 128
_NBUF = 3
_VECS = _D // _LANES
_SCALE = 0.5 / (2.0 * _BATCH)


def _sc_center_loss_partials(features, labels, centers):
    mesh = plsc.VectorSubcoreMesh(core_axis_name="c", subcore_axis_name="s")

    @functools.partial(
        pl.kernel,
        out_type=jax.ShapeDtypeStruct((_NW, _LANES), jnp.float32),
        mesh=mesh,
        scratch_types=[
            pltpu.VMEM((_ROWS_PER_W,), jnp.int32),
            pltpu.VMEM((_NBUF, _MAXCHUNK, _D), jnp.float32),
            pltpu.VMEM((_NBUF, _MAXCHUNK, _D), jnp.float32),
            pltpu.VMEM((_LANES,), jnp.float32),
        ] + [pltpu.SemaphoreType.DMA] * (2 * _NBUF),
    )
    def run(feat_hbm, idx_hbm, cent_hbm, out_hbm,
            idx_v, cent_v, feat_v, acc_v, *sems):
        wid = lax.axis_index("s") * _NC + lax.axis_index("c")
        base = wid * _ROWS_PER_W
        sg = sems[:_NBUF]
        sf = sems[_NBUF:]
        offs = []
        o = 0
        for n in _CHUNKS:
            offs.append(o)
            o += n

        def start(c):
            b = c % _NBUF
            n = _CHUNKS[c]
            o = offs[c]
            g = pltpu.async_copy(
                cent_hbm.at[idx_v.at[pl.ds(o, n)]],
                cent_v.at[b, pl.ds(0, n)], sg[b])
            f = pltpu.async_copy(
                feat_hbm.at[pl.ds(base + o, n)],
                feat_v.at[b, pl.ds(0, n)], sf[b])
            return g, f

        pltpu.sync_copy(idx_hbm.at[pl.ds(base, _ROWS_PER_W)], idx_v)
        pending = [start(c) for c in range(_NBUF)]
        acc = tuple(jnp.zeros((_LANES,), jnp.float32) for _ in range(_VECS))
        for c in range(len(_CHUNKS)):
            b = c % _NBUF
            g, f = pending[b]
            g.wait()
            f.wait()

            def body(r, a, b=b):
                out = []
                for v in range(_VECS):
                    fv = feat_v[b, r, pl.ds(v * _LANES, _LANES)]
                    cv = cent_v[b, r, pl.ds(v * _LANES, _LANES)]
                    d = fv - cv
                    out.append(a[v] + d * d)
                return tuple(out)

            acc = plsc.parallel_loop(0, _CHUNKS[c], unroll=4, carry=acc)(body)
            if c + _NBUF < len(_CHUNKS):
                pending[b] = start(c + _NBUF)

        tot = acc[0]
        for v in range(1, _VECS):
            tot = tot + acc[v]
        acc_v[...] = tot * jnp.float32(_SCALE)
        pltpu.sync_copy(acc_v, out_hbm.at[wid])

    return run(features, labels, centers)


def kernel(features, labels, centers):
    labels_i32 = labels.astype(jnp.int32)
    partials = _sc_center_loss_partials(features, labels_i32, centers)
    return jnp.sum(partials)


# speedup vs baseline: 1.0250x; 1.0015x over previous
import functools

import jax
import jax.numpy as jnp
from jax import lax
from jax.experimental import pallas as pl
from jax.experimental.pallas import tpu as pltpu
from jax.experimental.pallas import tpu_sc as plsc

_LANES = 16
_NC = 2
_NS = 16
_NW = _NC * _NS
_BATCH = 16384
_D = 128
_ROWS_PER_W = _BATCH // _NW
_CHUNKS = (32, 96, 128, 128, 96, 32)
_MAXCHUNK = 128
_NBUF = 3
_VECS = _D // _LANES
_SCALE = 0.5 / (2.0 * _BATCH)


def _sc_center_loss_partials(features, labels, centers):
    mesh = plsc.VectorSubcoreMesh(core_axis_name="c", subcore_axis_name="s")

    @functools.partial(
        pl.kernel,
        out_type=jax.ShapeDtypeStruct((_NW, _LANES), jnp.float32),
        mesh=mesh,
        scratch_types=[
            pltpu.VMEM((_ROWS_PER_W,), jnp.int32),
            pltpu.VMEM((_NBUF, _MAXCHUNK, _D), jnp.float32),
            pltpu.VMEM((_NBUF, _MAXCHUNK, _D), jnp.float32),
            pltpu.VMEM((_LANES,), jnp.float32),
        ] + [pltpu.SemaphoreType.DMA] * (2 * _NBUF),
    )
    def run(feat_hbm, idx_hbm, cent_hbm, out_hbm,
            idx_v, cent_v, feat_v, acc_v, *sems):
        wid = lax.axis_index("s") * _NC + lax.axis_index("c")
        base = wid * _ROWS_PER_W
        sg = sems[:_NBUF]
        sf = sems[_NBUF:]
        offs = []
        o = 0
        for n in _CHUNKS:
            offs.append(o)
            o += n

        def start(c):
            b = c % _NBUF
            n = _CHUNKS[c]
            o = offs[c]
            g = pltpu.async_copy(
                cent_hbm.at[idx_v.at[pl.ds(o, n)]],
                cent_v.at[b, pl.ds(0, n)], sg[b])
            f = pltpu.async_copy(
                feat_hbm.at[pl.ds(base + o, n)],
                feat_v.at[b, pl.ds(0, n)], sf[b])
            return g, f

        pltpu.sync_copy(idx_hbm.at[pl.ds(base, _ROWS_PER_W)], idx_v)
        pending = [start(c) for c in range(_NBUF)]
        acc = tuple(jnp.zeros((_LANES,), jnp.float32) for _ in range(_VECS))
        for c in range(len(_CHUNKS)):
            b = c % _NBUF
            g, f = pending[b]
            g.wait()
            f.wait()

            def body(r, a, b=b):
                out = []
                for v in range(_VECS):
                    fv = feat_v[b, r, pl.ds(v * _LANES, _LANES)]
                    cv = cent_v[b, r, pl.ds(v * _LANES, _LANES)]
                    d = fv - cv
                    out.append(a[v] + d * d)
                return tuple(out)

            acc = plsc.parallel_loop(0, _CHUNKS[c], unroll=8, carry=acc)(body)
            if c + _NBUF < len(_CHUNKS):
                pending[b] = start(c + _NBUF)

        tot = acc[0]
        for v in range(1, _VECS):
            tot = tot + acc[v]
        acc_v[...] = tot * jnp.float32(_SCALE)
        pltpu.sync_copy(acc_v, out_hbm.at[wid])

    return run(features, labels, centers)


def kernel(features, labels, centers):
    labels_i32 = labels.astype(jnp.int32)
    partials = _sc_center_loss_partials(features, labels_i32, centers)
    return jnp.sum(partials)
